# baseline (speedup 1.0000x reference)
import functools

import jax
import jax.numpy as jnp
from jax import lax
from jax.experimental import pallas as pl
from jax.experimental.pallas import tpu as pltpu, tpu_sc as plsc

N = 262144
NCOL = 66
OCOL = 13
NUMERIC = 10
BLOCKS = ((10, 18), (18, 34), (34, 66))

NC = 2
NS = 16
L = 16
NW = NC * NS
ROWS_PER_W = N // NW
CH = 512
NCHUNK = ROWS_PER_W // CH

_mesh = plsc.VectorSubcoreMesh(core_axis_name="c", subcore_axis_name="s")


@functools.partial(
    pl.kernel,
    out_type=jax.ShapeDtypeStruct((N * OCOL,), jnp.float32),
    mesh=_mesh,
    compiler_params=pltpu.CompilerParams(needs_layout_passes=False),
    scratch_types=[
        pltpu.VMEM((CH * NCOL,), jnp.float32),
        pltpu.VMEM((CH * OCOL,), jnp.float32),
    ],
)
def _onehot_to_numeric(x_hbm, out_hbm, inbuf, outbuf):
    wid = lax.axis_index("s") * NC + lax.axis_index("c")
    base_row = wid * ROWS_PER_W
    lane = lax.iota(jnp.int32, L)

    def group_body(g, carry):
        lr = g * L + lane
        srow = lr * NCOL
        orow = lr * OCOL
        for c in range(NUMERIC):
            v = plsc.load_gather(inbuf, [srow + c])
            plsc.store_scatter(outbuf, [orow + c], v)
        for slot, (s, e) in enumerate(BLOCKS):
            m = plsc.load_gather(inbuf, [srow + s])
            a = jnp.zeros((L,), jnp.float32)
            for j in range(1, e - s):
                v = plsc.load_gather(inbuf, [srow + s + j])
                upd = v > m
                m = jnp.where(upd, v, m)
                a = jnp.where(upd, jnp.float32(j), a)
            plsc.store_scatter(outbuf, [orow + NUMERIC + slot], a)
        return carry

    def chunk_body(ch, carry):
        r0 = base_row + ch * CH
        pltpu.sync_copy(x_hbm.at[pl.ds(r0 * NCOL, CH * NCOL)], inbuf)
        lax.fori_loop(0, CH // L, group_body, 0)
        pltpu.sync_copy(outbuf, out_hbm.at[pl.ds(r0 * OCOL, CH * OCOL)])
        return carry

    lax.fori_loop(0, NCHUNK, chunk_body, 0)


def kernel(X):
    out = _onehot_to_numeric(X.reshape(-1))
    return out.reshape(N, OCOL)
